# baseline (speedup 1.0000x reference)
import functools
import jax
import jax.numpy as jnp
from jax import lax
from jax.experimental import pallas as pl
from jax.experimental.pallas import tpu as pltpu
from jax.experimental.pallas import tpu_sc as plsc

VOCAB = 1000
MAXLEN = 512
WIDTH = VOCAB + MAXLEN
CB = 56
NBANDS = WIDTH // CB
NW = 32


def _sc_body(xt_hbm, out_hbm, xrow_v, buf, sem):
    s_len, b = 50, 1024
    nt = s_len * NBANDS
    kmax = (nt + NW - 1) // NW
    wid = lax.axis_index("s") * 2 + lax.axis_index("c")
    ones16 = jnp.full((16,), 1.0, jnp.float32)
    zeros16 = jnp.zeros((16,), jnp.float32)

    def zero_row(r, _):
        def zg(g, _):
            buf[r, pl.ds(g * 16, 16)] = zeros16
            return 0
        return lax.fori_loop(0, b // 16, zg, 0)

    lax.fori_loop(0, CB, zero_row, 0)

    def scatter_band(c0, val, msk_and):
        def sg(g, _):
            xv = xrow_v[pl.ds(g * 16, 16)]
            msk = (xv >= c0) & (xv < c0 + CB)
            b_idx = lax.broadcasted_iota(jnp.int32, (16,), 0) + g * 16
            row = jnp.where(msk, xv - c0, 0)
            plsc.store_scatter(buf, [row, b_idx], val, mask=msk)
            return 0
        lax.fori_loop(0, b // 16, sg, 0)

    def pos_row(pr, val):
        def pg(g, _):
            buf[pr, pl.ds(g * 16, 16)] = val
            return 0
        lax.fori_loop(0, b // 16, pg, 0)

    def task(k, _):
        t = wid + k * NW

        @pl.when(t < nt)
        def _():
            s = t // NBANDS
            band = t - s * NBANDS
            c0 = band * CB
            pltpu.sync_copy(xt_hbm.at[s], xrow_v)
            scatter_band(c0, ones16, True)
            pr = VOCAB + s - c0

            @pl.when((pr >= 0) & (pr < CB))
            def _():
                pos_row(pr, ones16)

            pltpu.async_copy(buf, out_hbm.at[s, pl.ds(c0, CB)], sem).wait()
            scatter_band(c0, zeros16, True)

            @pl.when((pr >= 0) & (pr < CB))
            def _():
                pos_row(pr, zeros16)

        return 0

    lax.fori_loop(0, kmax, task, 0)


def kernel(x):
    b, s = x.shape
    xt = x.T
    mesh = plsc.VectorSubcoreMesh(core_axis_name="c", subcore_axis_name="s")
    sck = functools.partial(
        pl.kernel,
        mesh=mesh,
        out_type=jax.ShapeDtypeStruct((s, WIDTH, b), jnp.float32),
        scratch_types=[
            pltpu.VMEM((b,), jnp.int32),
            pltpu.VMEM((CB, b), jnp.float32),
            pltpu.SemaphoreType.DMA,
        ],
        compiler_params=pltpu.CompilerParams(needs_layout_passes=False),
    )(_sc_body)
    out = sck(xt)
    return out.transpose(2, 0, 1)


# speedup vs baseline: 1.5382x; 1.5382x over previous
import functools
import jax
import jax.numpy as jnp
from jax import lax
from jax.experimental import pallas as pl
from jax.experimental.pallas import tpu as pltpu
from jax.experimental.pallas import tpu_sc as plsc

VOCAB = 1000
MAXLEN = 512
WIDTH = VOCAB + MAXLEN
CB = 56
NBANDS = WIDTH // CB
NW = 32


def _sc_body(xt_hbm, out_hbm, xr0, xr1, buf0, buf1, sem0, sem1):
    s_len, b = xt_hbm.shape
    nt = s_len * NBANDS
    base, rem = nt // NW, nt % NW
    wid = lax.axis_index("s") * 2 + lax.axis_index("c")
    t0 = wid * base + jnp.minimum(wid, rem)
    cnt = base + jnp.where(wid < rem, 1, 0)
    kmax = base + (1 if rem else 0)
    ones16 = jnp.full((16,), 1.0, jnp.float32)
    zeros16 = jnp.zeros((16,), jnp.float32)

    def zero_buf(buf):
        def zr(r, _):
            def zg(g, _):
                buf[r, pl.ds(g * 16, 16)] = zeros16
                return 0
            return lax.fori_loop(0, b // 16, zg, 0)
        lax.fori_loop(0, CB, zr, 0)

    zero_buf(buf0)
    zero_buf(buf1)

    def scatter_band(buf, xr, c0, val):
        def sg(g, _):
            xv = xr[pl.ds(g * 16, 16)]
            msk = (xv >= c0) & (xv < c0 + CB)
            b_idx = lax.broadcasted_iota(jnp.int32, (16,), 0) + g * 16
            row = jnp.where(msk, xv - c0, 0)
            plsc.store_scatter(buf, [row, b_idx], val, mask=msk)
            return 0
        lax.fori_loop(0, b // 16, sg, 0)

    def pos_row(buf, s, c0, val):
        pr = VOCAB + s - c0

        @pl.when((pr >= 0) & (pr < CB))
        def _():
            def pg(g, _):
                buf[pr, pl.ds(g * 16, 16)] = val
                return 0
            lax.fori_loop(0, b // 16, pg, 0)

    def step(k, buf, xr, sem):
        t = t0 + k
        s = t // NBANDS
        c0 = (t - s * NBANDS) * CB

        @pl.when(k >= 2)
        def _():
            tp = t - 2
            sp = tp // NBANDS
            cp = (tp - sp * NBANDS) * CB
            pltpu.make_async_copy(buf, out_hbm.at[sp, pl.ds(cp, CB)], sem).wait()
            scatter_band(buf, xr, cp, zeros16)
            pos_row(buf, sp, cp, zeros16)

            @pl.when(sp != s)
            def _():
                pltpu.sync_copy(xt_hbm.at[s], xr)

        @pl.when(k < 2)
        def _():
            pltpu.sync_copy(xt_hbm.at[s], xr)

        scatter_band(buf, xr, c0, ones16)
        pos_row(buf, s, c0, ones16)
        pltpu.async_copy(buf, out_hbm.at[s, pl.ds(c0, CB)], sem)

    def task(k, _):
        @pl.when(k < cnt)
        def _():
            @pl.when(k % 2 == 0)
            def _():
                step(k, buf0, xr0, sem0)

            @pl.when(k % 2 == 1)
            def _():
                step(k, buf1, xr1, sem1)

        return 0

    lax.fori_loop(0, kmax, task, 0)

    def drain(k, buf, sem):
        t = t0 + k
        s = t // NBANDS
        c0 = (t - s * NBANDS) * CB
        pltpu.make_async_copy(buf, out_hbm.at[s, pl.ds(c0, CB)], sem).wait()

    @pl.when(cnt >= 1)
    def _():
        k = cnt - 1

        @pl.when(k % 2 == 0)
        def _():
            drain(k, buf0, sem0)

        @pl.when(k % 2 == 1)
        def _():
            drain(k, buf1, sem1)

    @pl.when(cnt >= 2)
    def _():
        k = cnt - 2

        @pl.when(k % 2 == 0)
        def _():
            drain(k, buf0, sem0)

        @pl.when(k % 2 == 1)
        def _():
            drain(k, buf1, sem1)


def kernel(x):
    b, s = x.shape
    xt = x.T
    mesh = plsc.VectorSubcoreMesh(core_axis_name="c", subcore_axis_name="s")
    sck = functools.partial(
        pl.kernel,
        mesh=mesh,
        out_type=jax.ShapeDtypeStruct((s, WIDTH, b), jnp.float32),
        scratch_types=[
            pltpu.VMEM((b,), jnp.int32),
            pltpu.VMEM((b,), jnp.int32),
            pltpu.VMEM((CB, b), jnp.float32),
            pltpu.VMEM((CB, b), jnp.float32),
            pltpu.SemaphoreType.DMA,
            pltpu.SemaphoreType.DMA,
        ],
        compiler_params=pltpu.CompilerParams(needs_layout_passes=False),
    )(_sc_body)
    out = sck(xt)
    return out.transpose(2, 0, 1)
